# baseline (speedup 1.0000x reference)
import functools

import jax
import jax.numpy as jnp
from jax import lax
from jax.experimental import pallas as pl
from jax.experimental.pallas import tpu as pltpu
from jax.experimental.pallas import tpu_sc as plsc

N = 10000
E = 320000
D = 128

CH = 128
NS = 16
NCHUNK = 2560
PER_TILE = NCHUNK // NS
E_PAD = NCHUNK * CH - E
NPAD = 10240
PAD_DST = NPAD - N

ZERO_PER_TILE = NPAD // NS
CNT_CHUNK = 632
CNT_LAST = N - 15 * CNT_CHUNK

IBLK = 16

_MESH = plsc.VectorSubcoreMesh(core_axis_name="c", subcore_axis_name="s")


def _agg_body(with_counts, *refs):
    if with_counts:
        (src_ui, dst_ui, src_iu, dst_iu, tab_ui, tab_iu,
         aggr_i, aggr_u, cnt_i, cnt_u,
         idx_s, idx_d, rows, ones_v, zbuf,
         gsem0, gsem1, isem_s, isem_d, osem, ssem0, ssem1, acc, cnt_acc) = refs
    else:
        (src_ui, dst_ui, src_iu, dst_iu, tab_ui, tab_iu,
         aggr_i, aggr_u,
         idx_s, idx_d, rows, ones_v, zbuf,
         gsem0, gsem1, isem_s, isem_d, osem, ssem0, ssem1, acc, cnt_acc) = refs
        cnt_i = cnt_u = None

    c = lax.axis_index("c")
    s = lax.axis_index("s")

    def process(src_hbm, dst_hbm, tab_hbm, out_hbm, cnt_out_hbm):
        zv = jnp.zeros((16,), jnp.float32)

        def zrow(r, _):
            for j in range(8):
                rows[0, r, pl.ds(j * 16, 16)] = zv
            return 0
        lax.fori_loop(0, CH, zrow, 0)
        for k in range(ZERO_PER_TILE // CH):
            pltpu.sync_copy(rows.at[0],
                            acc.at[pl.ds(s * ZERO_PER_TILE + k * CH, CH)])
        if with_counts:
            def zflat(i, _):
                zbuf[pl.ds(i * 16, 16)] = zv
                return 0
            lax.fori_loop(0, ZERO_PER_TILE // 16, zflat, 0)
            for j in range(8):
                ones_v[pl.ds(j * 16, 16)] = jnp.ones((16,), jnp.float32)
            pltpu.sync_copy(zbuf,
                            cnt_acc.at[pl.ds(s * ZERO_PER_TILE,
                                             ZERO_PER_TILE)])

        plsc.subcore_barrier()

        start = s * PER_TILE
        NB = PER_TILE // IBLK

        def gath(p, k, buf, sem):
            return pltpu.async_copy(tab_hbm.at[idx_s.at[p, k]],
                                    rows.at[buf], sem)

        def scat(p, k, buf, sem):
            pltpu.async_copy(rows.at[buf], acc.at[idx_d.at[p, k]], sem,
                             add=True)
            if with_counts:
                pltpu.async_copy(ones_v, cnt_acc.at[idx_d.at[p, k]], osem,
                                 add=True)

        def scat_wait(buf, sem):
            pltpu.make_async_copy(rows.at[buf], acc.at[idx_d.at[0, 0]],
                                  sem).wait()

        pltpu.sync_copy(src_hbm.at[pl.ds(start, IBLK)], idx_s.at[0])
        pltpu.sync_copy(dst_hbm.at[pl.ds(start, IBLK)], idx_d.at[0])

        for b in range(NB):
            p = b % 2
            if b + 1 < NB:
                pn = (b + 1) % 2
                off = start + (b + 1) * IBLK
                pltpu.async_copy(src_hbm.at[pl.ds(off, IBLK)],
                                 idx_s.at[pn], isem_s)
                pltpu.async_copy(dst_hbm.at[pl.ds(off, IBLK)],
                                 idx_d.at[pn], isem_d)

            gath(p, 0, 0, gsem0)
            gath(p, 1, 1, gsem1)

            def gath_wait(buf, sem):
                pltpu.make_async_copy(tab_hbm.at[idx_s.at[p, 0]],
                                      rows.at[buf], sem).wait()

            def inner(i, _):
                gath_wait(0, gsem0)
                scat(p, 2 * i, 0, ssem0)
                gath_wait(1, gsem1)
                scat(p, 2 * i + 1, 1, ssem1)
                scat_wait(0, ssem0)
                gath(p, 2 * i + 2, 0, gsem0)
                scat_wait(1, ssem1)
                gath(p, 2 * i + 3, 1, gsem1)
                return 0
            lax.fori_loop(0, IBLK // 2 - 1, inner, 0)

            gath_wait(0, gsem0)
            scat(p, IBLK - 2, 0, ssem0)
            gath_wait(1, gsem1)
            scat(p, IBLK - 1, 1, ssem1)
            scat_wait(0, ssem0)
            scat_wait(1, ssem1)

            if with_counts:
                for _k in range(IBLK):
                    pltpu.make_async_copy(ones_v, cnt_acc.at[idx_d.at[p, 0]],
                                          osem).wait()

            if b + 1 < NB:
                pn = (b + 1) % 2
                off = start + (b + 1) * IBLK
                pltpu.make_async_copy(src_hbm.at[pl.ds(off, IBLK)],
                                      idx_s.at[pn], isem_s).wait()
                pltpu.make_async_copy(dst_hbm.at[pl.ds(off, IBLK)],
                                      idx_d.at[pn], isem_d).wait()

        plsc.subcore_barrier()

        @pl.when(s < 15)
        def _():
            pltpu.sync_copy(acc.at[pl.ds(s * CNT_CHUNK, CNT_CHUNK)],
                            out_hbm.at[pl.ds(s * CNT_CHUNK, CNT_CHUNK)])
            if with_counts:
                pltpu.sync_copy(cnt_acc.at[pl.ds(s * CNT_CHUNK, CNT_CHUNK)],
                                zbuf.at[pl.ds(0, CNT_CHUNK)])
                pltpu.sync_copy(zbuf.at[pl.ds(0, CNT_CHUNK)],
                                cnt_out_hbm.at[pl.ds(s * CNT_CHUNK, CNT_CHUNK)])

        @pl.when(s == 15)
        def _():
            pltpu.sync_copy(acc.at[pl.ds(15 * CNT_CHUNK, CNT_LAST)],
                            out_hbm.at[pl.ds(15 * CNT_CHUNK, CNT_LAST)])
            if with_counts:
                pltpu.sync_copy(cnt_acc.at[pl.ds(15 * CNT_CHUNK, CNT_LAST)],
                                zbuf.at[pl.ds(0, CNT_LAST)])
                pltpu.sync_copy(zbuf.at[pl.ds(0, CNT_LAST)],
                                cnt_out_hbm.at[pl.ds(15 * CNT_CHUNK, CNT_LAST)])

    @pl.when(c == 0)
    def _():
        process(src_ui, dst_ui, tab_ui, aggr_i, cnt_i)

    @pl.when(c == 1)
    def _():
        process(src_iu, dst_iu, tab_iu, aggr_u, cnt_u)


def _make_agg(with_counts):
    outs = [jax.ShapeDtypeStruct((N, D), jnp.float32),
            jax.ShapeDtypeStruct((N, D), jnp.float32)]
    if with_counts:
        outs += [jax.ShapeDtypeStruct((N,), jnp.float32),
                 jax.ShapeDtypeStruct((N,), jnp.float32)]
    return pl.kernel(
        functools.partial(_agg_body, with_counts),
        out_type=tuple(outs),
        mesh=_MESH,
        scratch_types=[
            pltpu.VMEM((2, IBLK, CH), jnp.int32),
            pltpu.VMEM((2, IBLK, CH), jnp.int32),
            pltpu.VMEM((2, CH, D), jnp.float32),
            pltpu.VMEM((CH,), jnp.float32),
            pltpu.VMEM((ZERO_PER_TILE,), jnp.float32),
            pltpu.SemaphoreType.DMA,
            pltpu.SemaphoreType.DMA,
            pltpu.SemaphoreType.DMA,
            pltpu.SemaphoreType.DMA,
            pltpu.SemaphoreType.DMA,
            pltpu.SemaphoreType.DMA,
            pltpu.SemaphoreType.DMA,
            pltpu.VMEM_SHARED((NPAD, D), jnp.float32),
            pltpu.VMEM_SHARED((NPAD,), jnp.float32),
        ],
    )


_agg_counts = _make_agg(True)
_agg_plain = _make_agg(False)


def _dot_t(a, w):
    return lax.dot_general(a, w, (((1,), (1,)), ((), ())),
                           preferred_element_type=jnp.float32,
                           precision=lax.Precision.HIGHEST)


_BLK = 2000
_GRID = N // _BLK


def _row_spec():
    return pl.BlockSpec((_BLK, D), lambda i: (i, 0))


def _w_spec():
    return pl.BlockSpec((D, D), lambda i: (0, 0))


def _b_spec():
    return pl.BlockSpec((1, D), lambda i: (0, 0))


def _cnt_spec():
    return pl.BlockSpec((_BLK, 1), lambda i: (i, 0))


def _proj_body(xu, xi, pu, pi, hu, hi):
    hu[...] = _dot_t(xu[...], pu[...])
    hi[...] = _dot_t(xi[...], pi[...])


_proj = pl.pallas_call(
    _proj_body,
    grid=(_GRID,),
    in_specs=[_row_spec(), _row_spec(), _w_spec(), _w_spec()],
    out_specs=[_row_spec(), _row_spec()],
    out_shape=[jax.ShapeDtypeStruct((N, D), jnp.float32)] * 2,
)


def _conv_mid_body(ai, ci, hi, wl_ui, bl_ui, wr_ui,
                   au, cu, hu, wl_iu, bl_iu, wr_iu, oi, ou):
    mean_i = ai[...] * (1.0 / jnp.maximum(ci[...], 1.0))
    oi[...] = jax.nn.relu(_dot_t(mean_i, wl_ui[...]) + bl_ui[...]
                          + _dot_t(hi[...], wr_ui[...])) + hi[...]
    mean_u = au[...] * (1.0 / jnp.maximum(cu[...], 1.0))
    ou[...] = jax.nn.relu(_dot_t(mean_u, wl_iu[...]) + bl_iu[...]
                          + _dot_t(hu[...], wr_iu[...])) + hu[...]


_conv_mid = pl.pallas_call(
    _conv_mid_body,
    grid=(_GRID,),
    in_specs=[_row_spec(), _cnt_spec(), _row_spec(), _w_spec(), _b_spec(),
              _w_spec(),
              _row_spec(), _cnt_spec(), _row_spec(), _w_spec(), _b_spec(),
              _w_spec()],
    out_specs=[_row_spec(), _row_spec()],
    out_shape=[jax.ShapeDtypeStruct((N, D), jnp.float32)] * 2,
)


def _conv_out_body(ai, ci, oi, wl_ui, bl_ui, wr_ui,
                   au, cu, ou, wl_iu, bl_iu, wr_iu, zu, zi):
    mean_i = ai[...] * (1.0 / jnp.maximum(ci[...], 1.0))
    p_i = _dot_t(mean_i, wl_ui[...]) + bl_ui[...] + _dot_t(oi[...], wr_ui[...])
    nrm_i = jnp.sqrt(jnp.sum(p_i * p_i, axis=1, keepdims=True))
    zi[...] = p_i / jnp.maximum(nrm_i, 1e-12)
    mean_u = au[...] * (1.0 / jnp.maximum(cu[...], 1.0))
    p_u = _dot_t(mean_u, wl_iu[...]) + bl_iu[...] + _dot_t(ou[...], wr_iu[...])
    nrm_u = jnp.sqrt(jnp.sum(p_u * p_u, axis=1, keepdims=True))
    zu[...] = p_u / jnp.maximum(nrm_u, 1e-12)


_conv_out = pl.pallas_call(
    _conv_out_body,
    grid=(_GRID,),
    in_specs=[_row_spec(), _cnt_spec(), _row_spec(), _w_spec(), _b_spec(),
              _w_spec(),
              _row_spec(), _cnt_spec(), _row_spec(), _w_spec(), _b_spec(),
              _w_spec()],
    out_specs=[_row_spec(), _row_spec()],
    out_shape=[jax.ShapeDtypeStruct((N, D), jnp.float32)] * 2,
)


def kernel(x_user, x_item, edge_index_ui, edge_index_iu, P_user, P_item,
           c1_ui_Wl, c1_ui_bl, c1_ui_Wr, c1_iu_Wl, c1_iu_bl, c1_iu_Wr,
           c2_ui_Wl, c2_ui_bl, c2_ui_Wr, c2_iu_Wl, c2_iu_bl, c2_iu_Wr):
    pad_src = jnp.arange(E_PAD, dtype=jnp.int32) % N
    pad_dst = N + jnp.arange(E_PAD, dtype=jnp.int32) % PAD_DST

    def prep(row, pad):
        return jnp.concatenate([row, pad]).reshape(NCHUNK, CH)

    src_ui = prep(edge_index_ui[0], pad_src)
    dst_ui = prep(edge_index_ui[1], pad_dst)
    src_iu = prep(edge_index_iu[0], pad_src)
    dst_iu = prep(edge_index_iu[1], pad_dst)

    h_u, h_i = _proj(x_user, x_item, P_user, P_item)

    aggr_i1, aggr_u1, cnt_i, cnt_u = _agg_counts(
        src_ui, dst_ui, src_iu, dst_iu, h_u, h_i)
    cnt_i2 = cnt_i.reshape(N, 1)
    cnt_u2 = cnt_u.reshape(N, 1)

    o_i, o_u = _conv_mid(
        aggr_i1, cnt_i2, h_i, c1_ui_Wl, c1_ui_bl.reshape(1, D), c1_ui_Wr,
        aggr_u1, cnt_u2, h_u, c1_iu_Wl, c1_iu_bl.reshape(1, D), c1_iu_Wr)

    aggr_i2, aggr_u2 = _agg_plain(
        src_ui, dst_ui, src_iu, dst_iu, o_u, o_i)

    out_u, out_i = _conv_out(
        aggr_i2, cnt_i2, o_i, c2_ui_Wl, c2_ui_bl.reshape(1, D), c2_ui_Wr,
        aggr_u2, cnt_u2, o_u, c2_iu_Wl, c2_iu_bl.reshape(1, D), c2_iu_Wr)

    return (out_u, out_i)


# speedup vs baseline: 1.2290x; 1.2290x over previous
import functools

import jax
import jax.numpy as jnp
from jax import lax
from jax.experimental import pallas as pl
from jax.experimental.pallas import tpu as pltpu
from jax.experimental.pallas import tpu_sc as plsc

N = 10000
E = 320000
D = 128

CH = 128
NS = 16
NCHUNK = 2560
PER_TILE = NCHUNK // NS
E_PAD = NCHUNK * CH - E
NPAD = 10240
PAD_DST = NPAD - N

ZERO_PER_TILE = NPAD // NS
CNT_CHUNK = 632
CNT_LAST = N - 15 * CNT_CHUNK

IBLK = 16

_MESH = plsc.VectorSubcoreMesh(core_axis_name="c", subcore_axis_name="s")


def _agg_body(with_counts, *refs):
    if with_counts:
        (src_ui, dst_ui, src_iu, dst_iu, tab_ui, tab_iu,
         aggr_i, aggr_u, cnt_i, cnt_u,
         idx_s, idx_d, rows, ones_v, zbuf,
         gsem0, gsem1, isem_s, isem_d, osem, ssem0, ssem1, acc, cnt_acc) = refs
    else:
        (src_ui, dst_ui, src_iu, dst_iu, tab_ui, tab_iu,
         aggr_i, aggr_u,
         idx_s, idx_d, rows, ones_v, zbuf,
         gsem0, gsem1, isem_s, isem_d, osem, ssem0, ssem1, acc, cnt_acc) = refs
        cnt_i = cnt_u = None

    c = lax.axis_index("c")
    s = lax.axis_index("s")

    def process(src_hbm, dst_hbm, tab_hbm, out_hbm, cnt_out_hbm):
        zv = jnp.zeros((16,), jnp.float32)

        def zrow(r, _):
            for j in range(8):
                rows[0, r, pl.ds(j * 16, 16)] = zv
            return 0
        lax.fori_loop(0, CH, zrow, 0)
        for k in range(ZERO_PER_TILE // CH):
            pltpu.sync_copy(rows.at[0],
                            acc.at[pl.ds(s * ZERO_PER_TILE + k * CH, CH)])
        if with_counts:
            def zflat(i, _):
                zbuf[pl.ds(i * 16, 16)] = zv
                return 0
            lax.fori_loop(0, ZERO_PER_TILE // 16, zflat, 0)
            for j in range(8):
                ones_v[pl.ds(j * 16, 16)] = jnp.ones((16,), jnp.float32)
            pltpu.sync_copy(zbuf,
                            cnt_acc.at[pl.ds(s * ZERO_PER_TILE,
                                             ZERO_PER_TILE)])

        plsc.subcore_barrier()

        start = s * PER_TILE
        NB = PER_TILE // IBLK

        def gath(p, k, buf, sem):
            return pltpu.async_copy(tab_hbm.at[idx_s.at[p, k]],
                                    rows.at[buf], sem)

        def scat(p, k, buf):
            pltpu.sync_copy(rows.at[buf], acc.at[idx_d.at[p, k]], add=True)
            if with_counts:
                pltpu.async_copy(ones_v, cnt_acc.at[idx_d.at[p, k]], osem,
                                 add=True)

        pltpu.sync_copy(src_hbm.at[pl.ds(start, IBLK)], idx_s.at[0])
        pltpu.sync_copy(dst_hbm.at[pl.ds(start, IBLK)], idx_d.at[0])

        for b in range(NB):
            p = b % 2
            if b + 1 < NB:
                pn = (b + 1) % 2
                off = start + (b + 1) * IBLK
                pltpu.async_copy(src_hbm.at[pl.ds(off, IBLK)],
                                 idx_s.at[pn], isem_s)
                pltpu.async_copy(dst_hbm.at[pl.ds(off, IBLK)],
                                 idx_d.at[pn], isem_d)

            gath(p, 0, 0, gsem0)

            def gath_wait(buf, sem):
                pltpu.make_async_copy(tab_hbm.at[idx_s.at[p, 0]],
                                      rows.at[buf], sem).wait()

            def inner(i, _):
                gath(p, 2 * i + 1, 1, gsem1)
                gath_wait(0, gsem0)
                scat(p, 2 * i, 0)
                gath(p, 2 * i + 2, 0, gsem0)
                gath_wait(1, gsem1)
                scat(p, 2 * i + 1, 1)
                return 0
            lax.fori_loop(0, IBLK // 2 - 1, inner, 0)

            gath(p, IBLK - 1, 1, gsem1)
            gath_wait(0, gsem0)
            scat(p, IBLK - 2, 0)
            gath_wait(1, gsem1)
            scat(p, IBLK - 1, 1)

            if with_counts:
                for _k in range(IBLK):
                    pltpu.make_async_copy(ones_v, cnt_acc.at[idx_d.at[p, 0]],
                                          osem).wait()

            if b + 1 < NB:
                pn = (b + 1) % 2
                off = start + (b + 1) * IBLK
                pltpu.make_async_copy(src_hbm.at[pl.ds(off, IBLK)],
                                      idx_s.at[pn], isem_s).wait()
                pltpu.make_async_copy(dst_hbm.at[pl.ds(off, IBLK)],
                                      idx_d.at[pn], isem_d).wait()

        plsc.subcore_barrier()

        @pl.when(s < 15)
        def _():
            pltpu.sync_copy(acc.at[pl.ds(s * CNT_CHUNK, CNT_CHUNK)],
                            out_hbm.at[pl.ds(s * CNT_CHUNK, CNT_CHUNK)])
            if with_counts:
                pltpu.sync_copy(cnt_acc.at[pl.ds(s * CNT_CHUNK, CNT_CHUNK)],
                                zbuf.at[pl.ds(0, CNT_CHUNK)])
                pltpu.sync_copy(zbuf.at[pl.ds(0, CNT_CHUNK)],
                                cnt_out_hbm.at[pl.ds(s * CNT_CHUNK, CNT_CHUNK)])

        @pl.when(s == 15)
        def _():
            pltpu.sync_copy(acc.at[pl.ds(15 * CNT_CHUNK, CNT_LAST)],
                            out_hbm.at[pl.ds(15 * CNT_CHUNK, CNT_LAST)])
            if with_counts:
                pltpu.sync_copy(cnt_acc.at[pl.ds(15 * CNT_CHUNK, CNT_LAST)],
                                zbuf.at[pl.ds(0, CNT_LAST)])
                pltpu.sync_copy(zbuf.at[pl.ds(0, CNT_LAST)],
                                cnt_out_hbm.at[pl.ds(15 * CNT_CHUNK, CNT_LAST)])

    @pl.when(c == 0)
    def _():
        process(src_ui, dst_ui, tab_ui, aggr_i, cnt_i)

    @pl.when(c == 1)
    def _():
        process(src_iu, dst_iu, tab_iu, aggr_u, cnt_u)


def _make_agg(with_counts):
    outs = [jax.ShapeDtypeStruct((N, D), jnp.float32),
            jax.ShapeDtypeStruct((N, D), jnp.float32)]
    if with_counts:
        outs += [jax.ShapeDtypeStruct((N,), jnp.float32),
                 jax.ShapeDtypeStruct((N,), jnp.float32)]
    return pl.kernel(
        functools.partial(_agg_body, with_counts),
        out_type=tuple(outs),
        mesh=_MESH,
        scratch_types=[
            pltpu.VMEM((2, IBLK, CH), jnp.int32),
            pltpu.VMEM((2, IBLK, CH), jnp.int32),
            pltpu.VMEM((2, CH, D), jnp.float32),
            pltpu.VMEM((CH,), jnp.float32),
            pltpu.VMEM((ZERO_PER_TILE,), jnp.float32),
            pltpu.SemaphoreType.DMA,
            pltpu.SemaphoreType.DMA,
            pltpu.SemaphoreType.DMA,
            pltpu.SemaphoreType.DMA,
            pltpu.SemaphoreType.DMA,
            pltpu.SemaphoreType.DMA,
            pltpu.SemaphoreType.DMA,
            pltpu.VMEM_SHARED((NPAD, D), jnp.float32),
            pltpu.VMEM_SHARED((NPAD,), jnp.float32),
        ],
    )


_agg_counts = _make_agg(True)
_agg_plain = _make_agg(False)


def _dot_t(a, w):
    return lax.dot_general(a, w, (((1,), (1,)), ((), ())),
                           preferred_element_type=jnp.float32)


_BLK = 2000
_GRID = N // _BLK


def _row_spec():
    return pl.BlockSpec((_BLK, D), lambda i: (i, 0))


def _w_spec():
    return pl.BlockSpec((D, D), lambda i: (0, 0))


def _b_spec():
    return pl.BlockSpec((1, D), lambda i: (0, 0))


def _cnt_spec():
    return pl.BlockSpec((_BLK, 1), lambda i: (i, 0))


def _proj_body(xu, xi, pu, pi, hu, hi):
    hu[...] = _dot_t(xu[...], pu[...])
    hi[...] = _dot_t(xi[...], pi[...])


_proj = pl.pallas_call(
    _proj_body,
    grid=(_GRID,),
    in_specs=[_row_spec(), _row_spec(), _w_spec(), _w_spec()],
    out_specs=[_row_spec(), _row_spec()],
    out_shape=[jax.ShapeDtypeStruct((N, D), jnp.float32)] * 2,
)


def _conv_mid_body(ai, ci, hi, wl_ui, bl_ui, wr_ui,
                   au, cu, hu, wl_iu, bl_iu, wr_iu, oi, ou):
    mean_i = ai[...] * (1.0 / jnp.maximum(ci[...], 1.0))
    oi[...] = jax.nn.relu(_dot_t(mean_i, wl_ui[...]) + bl_ui[...]
                          + _dot_t(hi[...], wr_ui[...])) + hi[...]
    mean_u = au[...] * (1.0 / jnp.maximum(cu[...], 1.0))
    ou[...] = jax.nn.relu(_dot_t(mean_u, wl_iu[...]) + bl_iu[...]
                          + _dot_t(hu[...], wr_iu[...])) + hu[...]


_conv_mid = pl.pallas_call(
    _conv_mid_body,
    grid=(_GRID,),
    in_specs=[_row_spec(), _cnt_spec(), _row_spec(), _w_spec(), _b_spec(),
              _w_spec(),
              _row_spec(), _cnt_spec(), _row_spec(), _w_spec(), _b_spec(),
              _w_spec()],
    out_specs=[_row_spec(), _row_spec()],
    out_shape=[jax.ShapeDtypeStruct((N, D), jnp.float32)] * 2,
)


def _conv_out_body(ai, ci, oi, wl_ui, bl_ui, wr_ui,
                   au, cu, ou, wl_iu, bl_iu, wr_iu, zu, zi):
    mean_i = ai[...] * (1.0 / jnp.maximum(ci[...], 1.0))
    p_i = _dot_t(mean_i, wl_ui[...]) + bl_ui[...] + _dot_t(oi[...], wr_ui[...])
    nrm_i = jnp.sqrt(jnp.sum(p_i * p_i, axis=1, keepdims=True))
    zi[...] = p_i / jnp.maximum(nrm_i, 1e-12)
    mean_u = au[...] * (1.0 / jnp.maximum(cu[...], 1.0))
    p_u = _dot_t(mean_u, wl_iu[...]) + bl_iu[...] + _dot_t(ou[...], wr_iu[...])
    nrm_u = jnp.sqrt(jnp.sum(p_u * p_u, axis=1, keepdims=True))
    zu[...] = p_u / jnp.maximum(nrm_u, 1e-12)


_conv_out = pl.pallas_call(
    _conv_out_body,
    grid=(_GRID,),
    in_specs=[_row_spec(), _cnt_spec(), _row_spec(), _w_spec(), _b_spec(),
              _w_spec(),
              _row_spec(), _cnt_spec(), _row_spec(), _w_spec(), _b_spec(),
              _w_spec()],
    out_specs=[_row_spec(), _row_spec()],
    out_shape=[jax.ShapeDtypeStruct((N, D), jnp.float32)] * 2,
)


def kernel(x_user, x_item, edge_index_ui, edge_index_iu, P_user, P_item,
           c1_ui_Wl, c1_ui_bl, c1_ui_Wr, c1_iu_Wl, c1_iu_bl, c1_iu_Wr,
           c2_ui_Wl, c2_ui_bl, c2_ui_Wr, c2_iu_Wl, c2_iu_bl, c2_iu_Wr):
    pad_src = jnp.arange(E_PAD, dtype=jnp.int32) % N
    pad_dst = N + jnp.arange(E_PAD, dtype=jnp.int32) % PAD_DST

    def prep(row, pad):
        return jnp.concatenate([row, pad]).reshape(NCHUNK, CH)

    src_ui = prep(edge_index_ui[0], pad_src)
    dst_ui = prep(edge_index_ui[1], pad_dst)
    src_iu = prep(edge_index_iu[0], pad_src)
    dst_iu = prep(edge_index_iu[1], pad_dst)

    h_u, h_i = _proj(x_user, x_item, P_user, P_item)

    aggr_i1, aggr_u1, cnt_i, cnt_u = _agg_counts(
        src_ui, dst_ui, src_iu, dst_iu, h_u, h_i)
    cnt_i2 = cnt_i.reshape(N, 1)
    cnt_u2 = cnt_u.reshape(N, 1)

    o_i, o_u = _conv_mid(
        aggr_i1, cnt_i2, h_i, c1_ui_Wl, c1_ui_bl.reshape(1, D), c1_ui_Wr,
        aggr_u1, cnt_u2, h_u, c1_iu_Wl, c1_iu_bl.reshape(1, D), c1_iu_Wr)

    aggr_i2, aggr_u2 = _agg_plain(
        src_ui, dst_ui, src_iu, dst_iu, o_u, o_i)

    out_u, out_i = _conv_out(
        aggr_i2, cnt_i2, o_i, c2_ui_Wl, c2_ui_bl.reshape(1, D), c2_ui_Wr,
        aggr_u2, cnt_u2, o_u, c2_iu_Wl, c2_iu_bl.reshape(1, D), c2_iu_Wr)

    return (out_u, out_i)


# speedup vs baseline: 1.2692x; 1.0327x over previous
import functools

import jax
import jax.numpy as jnp
from jax import lax
from jax.experimental import pallas as pl
from jax.experimental.pallas import tpu as pltpu
from jax.experimental.pallas import tpu_sc as plsc

N = 10000
E = 320000
D = 128

CH = 128
NS = 16
NCHUNK = 2560
PER_TILE = NCHUNK // NS
E_PAD = NCHUNK * CH - E
NPAD = 10240
PAD_DST = NPAD - N

ZERO_PER_TILE = NPAD // NS
CNT_CHUNK = 632
CNT_LAST = N - 15 * CNT_CHUNK

IBLK = 16

_MESH = plsc.VectorSubcoreMesh(core_axis_name="c", subcore_axis_name="s")


def _agg_body(with_counts, *refs):
    if with_counts:
        (src_ui, dst_ui, src_iu, dst_iu, tab_ui, tab_iu,
         aggr_i, aggr_u, cnt_i, cnt_u,
         idx_s, idx_d, rows, ones_v, zbuf,
         gsem0, gsem1, isem_s, isem_d, osem, ssem0, ssem1, acc, cnt_acc) = refs
    else:
        (src_ui, dst_ui, src_iu, dst_iu, tab_ui, tab_iu,
         aggr_i, aggr_u,
         idx_s, idx_d, rows, ones_v, zbuf,
         gsem0, gsem1, isem_s, isem_d, osem, ssem0, ssem1, acc, cnt_acc) = refs
        cnt_i = cnt_u = None

    c = lax.axis_index("c")
    s = lax.axis_index("s")

    def process(src_hbm, dst_hbm, tab_hbm, out_hbm, cnt_out_hbm):
        zv = jnp.zeros((16,), jnp.float32)

        def zrow(r, _):
            for j in range(8):
                rows[0, r, pl.ds(j * 16, 16)] = zv
            return 0
        lax.fori_loop(0, CH, zrow, 0)
        for k in range(ZERO_PER_TILE // CH):
            pltpu.async_copy(rows.at[0],
                             acc.at[pl.ds(s * ZERO_PER_TILE + k * CH, CH)],
                             gsem0)
        for k in range(ZERO_PER_TILE // CH):
            pltpu.make_async_copy(
                rows.at[0], acc.at[pl.ds(s * ZERO_PER_TILE, CH)],
                gsem0).wait()
        if with_counts:
            def zflat(i, _):
                zbuf[pl.ds(i * 16, 16)] = zv
                return 0
            lax.fori_loop(0, ZERO_PER_TILE // 16, zflat, 0)
            for j in range(8):
                ones_v[pl.ds(j * 16, 16)] = jnp.ones((16,), jnp.float32)
            pltpu.sync_copy(zbuf,
                            cnt_acc.at[pl.ds(s * ZERO_PER_TILE,
                                             ZERO_PER_TILE)])

        plsc.subcore_barrier()

        start = s * PER_TILE
        NB = PER_TILE // IBLK

        def gath(p, k, buf, sem):
            return pltpu.async_copy(tab_hbm.at[idx_s.at[p, k]],
                                    rows.at[buf], sem)

        def scat(p, k, buf):
            pltpu.sync_copy(rows.at[buf], acc.at[idx_d.at[p, k]], add=True)
            if with_counts:
                pltpu.async_copy(ones_v, cnt_acc.at[idx_d.at[p, k]], osem,
                                 add=True)

        pltpu.sync_copy(src_hbm.at[pl.ds(start, IBLK)], idx_s.at[0])
        pltpu.sync_copy(dst_hbm.at[pl.ds(start, IBLK)], idx_d.at[0])

        for b in range(NB):
            p = b % 2
            if b + 1 < NB:
                pn = (b + 1) % 2
                off = start + (b + 1) * IBLK
                pltpu.async_copy(src_hbm.at[pl.ds(off, IBLK)],
                                 idx_s.at[pn], isem_s)
                pltpu.async_copy(dst_hbm.at[pl.ds(off, IBLK)],
                                 idx_d.at[pn], isem_d)

            gath(p, 0, 0, gsem0)

            def gath_wait(buf, sem):
                pltpu.make_async_copy(tab_hbm.at[idx_s.at[p, 0]],
                                      rows.at[buf], sem).wait()

            def inner(i, _):
                gath(p, 2 * i + 1, 1, gsem1)
                gath_wait(0, gsem0)
                scat(p, 2 * i, 0)
                gath(p, 2 * i + 2, 0, gsem0)
                gath_wait(1, gsem1)
                scat(p, 2 * i + 1, 1)
                return 0
            lax.fori_loop(0, IBLK // 2 - 1, inner, 0)

            gath(p, IBLK - 1, 1, gsem1)
            gath_wait(0, gsem0)
            scat(p, IBLK - 2, 0)
            gath_wait(1, gsem1)
            scat(p, IBLK - 1, 1)

            if with_counts:
                for _k in range(IBLK):
                    pltpu.make_async_copy(ones_v, cnt_acc.at[idx_d.at[p, 0]],
                                          osem).wait()

            if b + 1 < NB:
                pn = (b + 1) % 2
                off = start + (b + 1) * IBLK
                pltpu.make_async_copy(src_hbm.at[pl.ds(off, IBLK)],
                                      idx_s.at[pn], isem_s).wait()
                pltpu.make_async_copy(dst_hbm.at[pl.ds(off, IBLK)],
                                      idx_d.at[pn], isem_d).wait()

        plsc.subcore_barrier()

        @pl.when(s < 15)
        def _():
            pltpu.sync_copy(acc.at[pl.ds(s * CNT_CHUNK, CNT_CHUNK)],
                            out_hbm.at[pl.ds(s * CNT_CHUNK, CNT_CHUNK)])
            if with_counts:
                pltpu.sync_copy(cnt_acc.at[pl.ds(s * CNT_CHUNK, CNT_CHUNK)],
                                zbuf.at[pl.ds(0, CNT_CHUNK)])
                pltpu.sync_copy(zbuf.at[pl.ds(0, CNT_CHUNK)],
                                cnt_out_hbm.at[pl.ds(s * CNT_CHUNK, CNT_CHUNK)])

        @pl.when(s == 15)
        def _():
            pltpu.sync_copy(acc.at[pl.ds(15 * CNT_CHUNK, CNT_LAST)],
                            out_hbm.at[pl.ds(15 * CNT_CHUNK, CNT_LAST)])
            if with_counts:
                pltpu.sync_copy(cnt_acc.at[pl.ds(15 * CNT_CHUNK, CNT_LAST)],
                                zbuf.at[pl.ds(0, CNT_LAST)])
                pltpu.sync_copy(zbuf.at[pl.ds(0, CNT_LAST)],
                                cnt_out_hbm.at[pl.ds(15 * CNT_CHUNK, CNT_LAST)])

    @pl.when(c == 0)
    def _():
        process(src_ui, dst_ui, tab_ui, aggr_i, cnt_i)

    @pl.when(c == 1)
    def _():
        process(src_iu, dst_iu, tab_iu, aggr_u, cnt_u)


def _make_agg(with_counts):
    outs = [jax.ShapeDtypeStruct((N, D), jnp.float32),
            jax.ShapeDtypeStruct((N, D), jnp.float32)]
    if with_counts:
        outs += [jax.ShapeDtypeStruct((N,), jnp.float32),
                 jax.ShapeDtypeStruct((N,), jnp.float32)]
    return pl.kernel(
        functools.partial(_agg_body, with_counts),
        out_type=tuple(outs),
        mesh=_MESH,
        scratch_types=[
            pltpu.VMEM((2, IBLK, CH), jnp.int32),
            pltpu.VMEM((2, IBLK, CH), jnp.int32),
            pltpu.VMEM((2, CH, D), jnp.float32),
            pltpu.VMEM((CH,), jnp.float32),
            pltpu.VMEM((ZERO_PER_TILE,), jnp.float32),
            pltpu.SemaphoreType.DMA,
            pltpu.SemaphoreType.DMA,
            pltpu.SemaphoreType.DMA,
            pltpu.SemaphoreType.DMA,
            pltpu.SemaphoreType.DMA,
            pltpu.SemaphoreType.DMA,
            pltpu.SemaphoreType.DMA,
            pltpu.VMEM_SHARED((NPAD, D), jnp.float32),
            pltpu.VMEM_SHARED((NPAD,), jnp.float32),
        ],
    )


_agg_counts = _make_agg(True)
_agg_plain = _make_agg(False)


def _dot_t(a, w):
    return lax.dot_general(a, w, (((1,), (1,)), ((), ())),
                           preferred_element_type=jnp.float32)


_BLK = 2000
_GRID = N // _BLK


def _row_spec():
    return pl.BlockSpec((_BLK, D), lambda i: (i, 0))


def _w_spec():
    return pl.BlockSpec((D, D), lambda i: (0, 0))


def _b_spec():
    return pl.BlockSpec((1, D), lambda i: (0, 0))


def _cnt_spec():
    return pl.BlockSpec((_BLK, 1), lambda i: (i, 0))


_ECHUNK = NCHUNK // _GRID
_EROWS = E // CH


def _proj_body(xu, xi, pu, pi, eui, eiu, hu, hi, sui, dui, siu, diu):
    hu[...] = _dot_t(xu[...], pu[...])
    hi[...] = _dot_t(xi[...], pi[...])
    i = pl.program_id(0)
    row = i * _ECHUNK + jax.lax.broadcasted_iota(jnp.int32, (_ECHUNK, CH), 0)
    lane = jax.lax.broadcasted_iota(jnp.int32, (_ECHUNK, CH), 1)
    e = row * CH + lane
    valid = row < _EROWS
    pad_src = (e - E) % N
    pad_dst = N + (e - E) % PAD_DST
    sui[...] = jnp.where(valid, eui[0], pad_src)
    dui[...] = jnp.where(valid, eui[1], pad_dst)
    siu[...] = jnp.where(valid, eiu[0], pad_src)
    diu[...] = jnp.where(valid, eiu[1], pad_dst)


def _e_spec():
    return pl.BlockSpec((2, _ECHUNK, CH), lambda i: (0, i, 0))


def _eo_spec():
    return pl.BlockSpec((_ECHUNK, CH), lambda i: (i, 0))


_proj = pl.pallas_call(
    _proj_body,
    grid=(_GRID,),
    in_specs=[_row_spec(), _row_spec(), _w_spec(), _w_spec(),
              _e_spec(), _e_spec()],
    out_specs=[_row_spec(), _row_spec(),
               _eo_spec(), _eo_spec(), _eo_spec(), _eo_spec()],
    out_shape=[jax.ShapeDtypeStruct((N, D), jnp.float32)] * 2
    + [jax.ShapeDtypeStruct((NCHUNK, CH), jnp.int32)] * 4,
)


def _conv_mid_body(ai, ci, hi, wl_ui, bl_ui, wr_ui,
                   au, cu, hu, wl_iu, bl_iu, wr_iu, oi, ou):
    mean_i = ai[...] * (1.0 / jnp.maximum(ci[...], 1.0))
    oi[...] = jax.nn.relu(_dot_t(mean_i, wl_ui[...]) + bl_ui[...]
                          + _dot_t(hi[...], wr_ui[...])) + hi[...]
    mean_u = au[...] * (1.0 / jnp.maximum(cu[...], 1.0))
    ou[...] = jax.nn.relu(_dot_t(mean_u, wl_iu[...]) + bl_iu[...]
                          + _dot_t(hu[...], wr_iu[...])) + hu[...]


_conv_mid = pl.pallas_call(
    _conv_mid_body,
    grid=(_GRID,),
    in_specs=[_row_spec(), _cnt_spec(), _row_spec(), _w_spec(), _b_spec(),
              _w_spec(),
              _row_spec(), _cnt_spec(), _row_spec(), _w_spec(), _b_spec(),
              _w_spec()],
    out_specs=[_row_spec(), _row_spec()],
    out_shape=[jax.ShapeDtypeStruct((N, D), jnp.float32)] * 2,
)


def _conv_out_body(ai, ci, oi, wl_ui, bl_ui, wr_ui,
                   au, cu, ou, wl_iu, bl_iu, wr_iu, zu, zi):
    mean_i = ai[...] * (1.0 / jnp.maximum(ci[...], 1.0))
    p_i = _dot_t(mean_i, wl_ui[...]) + bl_ui[...] + _dot_t(oi[...], wr_ui[...])
    nrm_i = jnp.sqrt(jnp.sum(p_i * p_i, axis=1, keepdims=True))
    zi[...] = p_i / jnp.maximum(nrm_i, 1e-12)
    mean_u = au[...] * (1.0 / jnp.maximum(cu[...], 1.0))
    p_u = _dot_t(mean_u, wl_iu[...]) + bl_iu[...] + _dot_t(ou[...], wr_iu[...])
    nrm_u = jnp.sqrt(jnp.sum(p_u * p_u, axis=1, keepdims=True))
    zu[...] = p_u / jnp.maximum(nrm_u, 1e-12)


_conv_out = pl.pallas_call(
    _conv_out_body,
    grid=(_GRID,),
    in_specs=[_row_spec(), _cnt_spec(), _row_spec(), _w_spec(), _b_spec(),
              _w_spec(),
              _row_spec(), _cnt_spec(), _row_spec(), _w_spec(), _b_spec(),
              _w_spec()],
    out_specs=[_row_spec(), _row_spec()],
    out_shape=[jax.ShapeDtypeStruct((N, D), jnp.float32)] * 2,
)


def kernel(x_user, x_item, edge_index_ui, edge_index_iu, P_user, P_item,
           c1_ui_Wl, c1_ui_bl, c1_ui_Wr, c1_iu_Wl, c1_iu_bl, c1_iu_Wr,
           c2_ui_Wl, c2_ui_bl, c2_ui_Wr, c2_iu_Wl, c2_iu_bl, c2_iu_Wr):
    h_u, h_i, src_ui, dst_ui, src_iu, dst_iu = _proj(
        x_user, x_item, P_user, P_item,
        edge_index_ui.reshape(2, _EROWS, CH),
        edge_index_iu.reshape(2, _EROWS, CH))

    aggr_i1, aggr_u1, cnt_i, cnt_u = _agg_counts(
        src_ui, dst_ui, src_iu, dst_iu, h_u, h_i)
    cnt_i2 = cnt_i.reshape(N, 1)
    cnt_u2 = cnt_u.reshape(N, 1)

    o_i, o_u = _conv_mid(
        aggr_i1, cnt_i2, h_i, c1_ui_Wl, c1_ui_bl.reshape(1, D), c1_ui_Wr,
        aggr_u1, cnt_u2, h_u, c1_iu_Wl, c1_iu_bl.reshape(1, D), c1_iu_Wr)

    aggr_i2, aggr_u2 = _agg_plain(
        src_ui, dst_ui, src_iu, dst_iu, o_u, o_i)

    out_u, out_i = _conv_out(
        aggr_i2, cnt_i2, o_i, c2_ui_Wl, c2_ui_bl.reshape(1, D), c2_ui_Wr,
        aggr_u2, cnt_u2, o_u, c2_iu_Wl, c2_iu_bl.reshape(1, D), c2_iu_Wr)

    return (out_u, out_i)


# speedup vs baseline: 1.2708x; 1.0013x over previous
import functools

import jax
import jax.numpy as jnp
from jax import lax
from jax.experimental import pallas as pl
from jax.experimental.pallas import tpu as pltpu
from jax.experimental.pallas import tpu_sc as plsc

N = 10000
E = 320000
D = 128

CH = 128
NS = 16
NCHUNK = 2560
PER_TILE = NCHUNK // NS
NPAD = 10240
PAD_DST = NPAD - N

ZERO_PER_TILE = NPAD // NS
CNT_CHUNK = 632
CNT_LAST = N - 15 * CNT_CHUNK

IBLK = 16

_MESH = plsc.VectorSubcoreMesh(core_axis_name="c", subcore_axis_name="s")


def _agg_body(with_counts, *refs):
    if with_counts:
        (src_ui, dst_ui, src_iu, dst_iu, tab_ui, tab_iu,
         aggr_i, aggr_u, cnt_i, cnt_u,
         idx_s, idx_d, rows, ones_v, zbuf,
         gsem0, gsem1, isem_s, isem_d, osem, acc, cnt_acc) = refs
    else:
        (src_ui, dst_ui, src_iu, dst_iu, tab_ui, tab_iu,
         aggr_i, aggr_u,
         idx_s, idx_d, rows, ones_v, zbuf,
         gsem0, gsem1, isem_s, isem_d, osem, acc, cnt_acc) = refs
        cnt_i = cnt_u = None

    c = lax.axis_index("c")
    s = lax.axis_index("s")

    def process(src_hbm, dst_hbm, tab_hbm, out_hbm, cnt_out_hbm):
        zv = jnp.zeros((16,), jnp.float32)

        def zrow(r, _):
            for j in range(8):
                rows[0, r, pl.ds(j * 16, 16)] = zv
            return 0
        lax.fori_loop(0, CH, zrow, 0)
        for k in range(ZERO_PER_TILE // CH):
            pltpu.async_copy(rows.at[0],
                             acc.at[pl.ds(s * ZERO_PER_TILE + k * CH, CH)],
                             gsem0)
        for k in range(ZERO_PER_TILE // CH):
            pltpu.make_async_copy(
                rows.at[0], acc.at[pl.ds(s * ZERO_PER_TILE, CH)],
                gsem0).wait()
        if with_counts:
            def zflat(i, _):
                zbuf[pl.ds(i * 16, 16)] = zv
                return 0
            lax.fori_loop(0, ZERO_PER_TILE // 16, zflat, 0)
            for j in range(8):
                ones_v[pl.ds(j * 16, 16)] = jnp.ones((16,), jnp.float32)
            pltpu.sync_copy(zbuf,
                            cnt_acc.at[pl.ds(s * ZERO_PER_TILE,
                                             ZERO_PER_TILE)])

        plsc.subcore_barrier()

        start = s * PER_TILE
        NB = PER_TILE // IBLK

        def gath(p, k, buf, sem):
            return pltpu.async_copy(tab_hbm.at[idx_s.at[p, k]],
                                    rows.at[buf], sem)

        def scat(p, k, buf):
            pltpu.sync_copy(rows.at[buf], acc.at[idx_d.at[p, k]], add=True)
            if with_counts:
                pltpu.async_copy(ones_v, cnt_acc.at[idx_d.at[p, k]], osem,
                                 add=True)

        pltpu.sync_copy(src_hbm.at[pl.ds(start, IBLK)], idx_s.at[0])
        pltpu.sync_copy(dst_hbm.at[pl.ds(start, IBLK)], idx_d.at[0])

        for b in range(NB):
            p = b % 2
            if b + 1 < NB:
                pn = (b + 1) % 2
                off = start + (b + 1) * IBLK
                pltpu.async_copy(src_hbm.at[pl.ds(off, IBLK)],
                                 idx_s.at[pn], isem_s)
                pltpu.async_copy(dst_hbm.at[pl.ds(off, IBLK)],
                                 idx_d.at[pn], isem_d)

            gath(p, 0, 0, gsem0)

            def gath_wait(buf, sem):
                pltpu.make_async_copy(tab_hbm.at[idx_s.at[p, 0]],
                                      rows.at[buf], sem).wait()

            def inner(i, _):
                gath(p, 2 * i + 1, 1, gsem1)
                gath_wait(0, gsem0)
                scat(p, 2 * i, 0)
                gath(p, 2 * i + 2, 0, gsem0)
                gath_wait(1, gsem1)
                scat(p, 2 * i + 1, 1)
                return 0
            lax.fori_loop(0, IBLK // 2 - 1, inner, 0)

            gath(p, IBLK - 1, 1, gsem1)
            gath_wait(0, gsem0)
            scat(p, IBLK - 2, 0)
            gath_wait(1, gsem1)
            scat(p, IBLK - 1, 1)

            if with_counts:
                for _k in range(IBLK):
                    pltpu.make_async_copy(ones_v, cnt_acc.at[idx_d.at[p, 0]],
                                          osem).wait()

            if b + 1 < NB:
                pn = (b + 1) % 2
                off = start + (b + 1) * IBLK
                pltpu.make_async_copy(src_hbm.at[pl.ds(off, IBLK)],
                                      idx_s.at[pn], isem_s).wait()
                pltpu.make_async_copy(dst_hbm.at[pl.ds(off, IBLK)],
                                      idx_d.at[pn], isem_d).wait()

        plsc.subcore_barrier()

        @pl.when(s < 15)
        def _():
            pltpu.sync_copy(acc.at[pl.ds(s * CNT_CHUNK, CNT_CHUNK)],
                            out_hbm.at[pl.ds(s * CNT_CHUNK, CNT_CHUNK)])
            if with_counts:
                pltpu.sync_copy(cnt_acc.at[pl.ds(s * CNT_CHUNK, CNT_CHUNK)],
                                zbuf.at[pl.ds(0, CNT_CHUNK)])
                pltpu.sync_copy(zbuf.at[pl.ds(0, CNT_CHUNK)],
                                cnt_out_hbm.at[pl.ds(s * CNT_CHUNK, CNT_CHUNK)])

        @pl.when(s == 15)
        def _():
            pltpu.sync_copy(acc.at[pl.ds(15 * CNT_CHUNK, CNT_LAST)],
                            out_hbm.at[pl.ds(15 * CNT_CHUNK, CNT_LAST)])
            if with_counts:
                pltpu.sync_copy(cnt_acc.at[pl.ds(15 * CNT_CHUNK, CNT_LAST)],
                                zbuf.at[pl.ds(0, CNT_LAST)])
                pltpu.sync_copy(zbuf.at[pl.ds(0, CNT_LAST)],
                                cnt_out_hbm.at[pl.ds(15 * CNT_CHUNK, CNT_LAST)])

    @pl.when(c == 0)
    def _():
        process(src_ui, dst_ui, tab_ui, aggr_i, cnt_i)

    @pl.when(c == 1)
    def _():
        process(src_iu, dst_iu, tab_iu, aggr_u, cnt_u)


def _make_agg(with_counts):
    outs = [jax.ShapeDtypeStruct((N, D), jnp.float32),
            jax.ShapeDtypeStruct((N, D), jnp.float32)]
    if with_counts:
        outs += [jax.ShapeDtypeStruct((N,), jnp.float32),
                 jax.ShapeDtypeStruct((N,), jnp.float32)]
    return pl.kernel(
        functools.partial(_agg_body, with_counts),
        out_type=tuple(outs),
        mesh=_MESH,
        scratch_types=[
            pltpu.VMEM((2, IBLK, CH), jnp.int32),
            pltpu.VMEM((2, IBLK, CH), jnp.int32),
            pltpu.VMEM((2, CH, D), jnp.float32),
            pltpu.VMEM((CH,), jnp.float32),
            pltpu.VMEM((ZERO_PER_TILE,), jnp.float32),
            pltpu.SemaphoreType.DMA,
            pltpu.SemaphoreType.DMA,
            pltpu.SemaphoreType.DMA,
            pltpu.SemaphoreType.DMA,
            pltpu.SemaphoreType.DMA,
            pltpu.VMEM_SHARED((NPAD, D), jnp.float32),
            pltpu.VMEM_SHARED((NPAD,), jnp.float32),
        ],
    )


_agg_counts = _make_agg(True)
_agg_plain = _make_agg(False)


def _dot_t(a, w):
    return lax.dot_general(a, w, (((1,), (1,)), ((), ())),
                           preferred_element_type=jnp.float32)


_BLK = 2000
_GRID = N // _BLK


def _row_spec():
    return pl.BlockSpec((_BLK, D), lambda i: (i, 0))


def _w_spec():
    return pl.BlockSpec((D, D), lambda i: (0, 0))


def _b_spec():
    return pl.BlockSpec((1, D), lambda i: (0, 0))


def _cnt_spec():
    return pl.BlockSpec((_BLK, 1), lambda i: (i, 0))


_ECHUNK = NCHUNK // _GRID
_EROWS = E // CH


def _proj_body(xu, xi, pu, pi, eui, eiu, hu, hi, sui, dui, siu, diu):
    hu[...] = _dot_t(xu[...], pu[...])
    hi[...] = _dot_t(xi[...], pi[...])
    i = pl.program_id(0)
    row = i * _ECHUNK + jax.lax.broadcasted_iota(jnp.int32, (_ECHUNK, CH), 0)
    lane = jax.lax.broadcasted_iota(jnp.int32, (_ECHUNK, CH), 1)
    e = row * CH + lane
    valid = row < _EROWS
    pad_src = (e - E) % N
    pad_dst = N + (e - E) % PAD_DST
    sui[...] = jnp.where(valid, eui[0], pad_src)
    dui[...] = jnp.where(valid, eui[1], pad_dst)
    siu[...] = jnp.where(valid, eiu[0], pad_src)
    diu[...] = jnp.where(valid, eiu[1], pad_dst)


def _e_spec():
    return pl.BlockSpec((2, _ECHUNK, CH), lambda i: (0, i, 0))


def _eo_spec():
    return pl.BlockSpec((_ECHUNK, CH), lambda i: (i, 0))


_proj = pl.pallas_call(
    _proj_body,
    grid=(_GRID,),
    in_specs=[_row_spec(), _row_spec(), _w_spec(), _w_spec(),
              _e_spec(), _e_spec()],
    out_specs=[_row_spec(), _row_spec(),
               _eo_spec(), _eo_spec(), _eo_spec(), _eo_spec()],
    out_shape=[jax.ShapeDtypeStruct((N, D), jnp.float32)] * 2
    + [jax.ShapeDtypeStruct((NCHUNK, CH), jnp.int32)] * 4,
)


def _conv_mid_body(ai, ci, hi, wl_ui, bl_ui, wr_ui,
                   au, cu, hu, wl_iu, bl_iu, wr_iu, oi, ou):
    mean_i = ai[...] * (1.0 / jnp.maximum(ci[...], 1.0))
    oi[...] = jax.nn.relu(_dot_t(mean_i, wl_ui[...]) + bl_ui[...]
                          + _dot_t(hi[...], wr_ui[...])) + hi[...]
    mean_u = au[...] * (1.0 / jnp.maximum(cu[...], 1.0))
    ou[...] = jax.nn.relu(_dot_t(mean_u, wl_iu[...]) + bl_iu[...]
                          + _dot_t(hu[...], wr_iu[...])) + hu[...]


_conv_mid = pl.pallas_call(
    _conv_mid_body,
    grid=(_GRID,),
    in_specs=[_row_spec(), _cnt_spec(), _row_spec(), _w_spec(), _b_spec(),
              _w_spec(),
              _row_spec(), _cnt_spec(), _row_spec(), _w_spec(), _b_spec(),
              _w_spec()],
    out_specs=[_row_spec(), _row_spec()],
    out_shape=[jax.ShapeDtypeStruct((N, D), jnp.float32)] * 2,
)


def _conv_out_body(ai, ci, oi, wl_ui, bl_ui, wr_ui,
                   au, cu, ou, wl_iu, bl_iu, wr_iu, zu, zi):
    mean_i = ai[...] * (1.0 / jnp.maximum(ci[...], 1.0))
    p_i = _dot_t(mean_i, wl_ui[...]) + bl_ui[...] + _dot_t(oi[...], wr_ui[...])
    nrm_i = jnp.sqrt(jnp.sum(p_i * p_i, axis=1, keepdims=True))
    zi[...] = p_i / jnp.maximum(nrm_i, 1e-12)
    mean_u = au[...] * (1.0 / jnp.maximum(cu[...], 1.0))
    p_u = _dot_t(mean_u, wl_iu[...]) + bl_iu[...] + _dot_t(ou[...], wr_iu[...])
    nrm_u = jnp.sqrt(jnp.sum(p_u * p_u, axis=1, keepdims=True))
    zu[...] = p_u / jnp.maximum(nrm_u, 1e-12)


_conv_out = pl.pallas_call(
    _conv_out_body,
    grid=(_GRID,),
    in_specs=[_row_spec(), _cnt_spec(), _row_spec(), _w_spec(), _b_spec(),
              _w_spec(),
              _row_spec(), _cnt_spec(), _row_spec(), _w_spec(), _b_spec(),
              _w_spec()],
    out_specs=[_row_spec(), _row_spec()],
    out_shape=[jax.ShapeDtypeStruct((N, D), jnp.float32)] * 2,
)


def kernel(x_user, x_item, edge_index_ui, edge_index_iu, P_user, P_item,
           c1_ui_Wl, c1_ui_bl, c1_ui_Wr, c1_iu_Wl, c1_iu_bl, c1_iu_Wr,
           c2_ui_Wl, c2_ui_bl, c2_ui_Wr, c2_iu_Wl, c2_iu_bl, c2_iu_Wr):
    h_u, h_i, src_ui, dst_ui, src_iu, dst_iu = _proj(
        x_user, x_item, P_user, P_item,
        edge_index_ui.reshape(2, _EROWS, CH),
        edge_index_iu.reshape(2, _EROWS, CH))

    aggr_i1, aggr_u1, cnt_i, cnt_u = _agg_counts(
        src_ui, dst_ui, src_iu, dst_iu, h_u, h_i)
    cnt_i2 = cnt_i.reshape(N, 1)
    cnt_u2 = cnt_u.reshape(N, 1)

    o_i, o_u = _conv_mid(
        aggr_i1, cnt_i2, h_i, c1_ui_Wl, c1_ui_bl.reshape(1, D), c1_ui_Wr,
        aggr_u1, cnt_u2, h_u, c1_iu_Wl, c1_iu_bl.reshape(1, D), c1_iu_Wr)

    aggr_i2, aggr_u2 = _agg_plain(
        src_ui, dst_ui, src_iu, dst_iu, o_u, o_i)

    out_u, out_i = _conv_out(
        aggr_i2, cnt_i2, o_i, c2_ui_Wl, c2_ui_bl.reshape(1, D), c2_ui_Wr,
        aggr_u2, cnt_u2, o_u, c2_iu_Wl, c2_iu_bl.reshape(1, D), c2_iu_Wr)

    return (out_u, out_i)


# speedup vs baseline: 1.3336x; 1.0494x over previous
import functools

import jax
import jax.numpy as jnp
from jax import lax
from jax.experimental import pallas as pl
from jax.experimental.pallas import tpu as pltpu
from jax.experimental.pallas import tpu_sc as plsc

N = 10000
E = 320000
D = 128

CH = 128
NS = 16
NCHUNK = 2560
PER_TILE = NCHUNK // NS
NPAD = 10240
PAD_DST = NPAD - N

ZERO_PER_TILE = NPAD // NS
CNT_CHUNK = 632
CNT_LAST = N - 15 * CNT_CHUNK

IBLK = 16

_MESH = plsc.VectorSubcoreMesh(core_axis_name="c", subcore_axis_name="s")


def _agg_body(with_counts, *refs):
    if with_counts:
        (src_ui, dst_ui, src_iu, dst_iu, tab_ui, tab_iu,
         aggr_i, aggr_u, cnt_i, cnt_u,
         idx_s, idx_d, rows, ones_v, zbuf,
         gsem0, gsem1, isem_s, isem_d, osem, acc, cnt_acc) = refs
    else:
        (src_ui, dst_ui, src_iu, dst_iu, tab_ui, tab_iu,
         aggr_i, aggr_u,
         idx_s, idx_d, rows, ones_v, zbuf,
         gsem0, gsem1, isem_s, isem_d, osem, acc, cnt_acc) = refs
        cnt_i = cnt_u = None

    c = lax.axis_index("c")
    s = lax.axis_index("s")

    def process(src_hbm, dst_hbm, tab_hbm, out_hbm, cnt_out_hbm):
        zv = jnp.zeros((16,), jnp.float32)

        def zrow(r, _):
            for j in range(8):
                rows[0, r, pl.ds(j * 16, 16)] = zv
            return 0
        lax.fori_loop(0, CH, zrow, 0)
        for k in range(ZERO_PER_TILE // CH):
            pltpu.async_copy(rows.at[0],
                             acc.at[pl.ds(s * ZERO_PER_TILE + k * CH, CH)],
                             gsem0)
        for k in range(ZERO_PER_TILE // CH):
            pltpu.make_async_copy(
                rows.at[0], acc.at[pl.ds(s * ZERO_PER_TILE, CH)],
                gsem0).wait()
        if with_counts:
            def zflat(i, _):
                zbuf[pl.ds(i * 16, 16)] = zv
                return 0
            lax.fori_loop(0, ZERO_PER_TILE // 16, zflat, 0)
            for j in range(8):
                ones_v[pl.ds(j * 16, 16)] = jnp.ones((16,), jnp.float32)
            pltpu.sync_copy(zbuf,
                            cnt_acc.at[pl.ds(s * ZERO_PER_TILE,
                                             ZERO_PER_TILE)])

        plsc.subcore_barrier()

        start = s * PER_TILE
        NB = PER_TILE // IBLK

        def gath(p, k, buf, sem):
            return pltpu.async_copy(tab_hbm.at[idx_s.at[p, k]],
                                    rows.at[buf], sem)

        def scat(p, k, buf):
            pltpu.sync_copy(rows.at[buf], acc.at[idx_d.at[p, k]], add=True)
            if with_counts:
                pltpu.async_copy(ones_v, cnt_acc.at[idx_d.at[p, k]], osem,
                                 add=True)

        def gath_wait(buf, sem):
            pltpu.make_async_copy(tab_hbm.at[idx_s.at[0, 0]],
                                  rows.at[buf], sem).wait()

        pltpu.sync_copy(src_hbm.at[pl.ds(start, IBLK)], idx_s.at[0])
        pltpu.sync_copy(dst_hbm.at[pl.ds(start, IBLK)], idx_d.at[0])
        gath(0, 0, 0, gsem0)

        for b in range(NB):
            p = b % 2
            pn = (b + 1) % 2
            off = start + (b + 1) * IBLK
            if b + 1 < NB:
                pltpu.async_copy(src_hbm.at[pl.ds(off, IBLK)],
                                 idx_s.at[pn], isem_s)
                pltpu.async_copy(dst_hbm.at[pl.ds(off, IBLK)],
                                 idx_d.at[pn], isem_d)

            def inner(i, _):
                gath(p, 2 * i + 1, 1, gsem1)
                gath_wait(0, gsem0)
                scat(p, 2 * i, 0)
                gath(p, 2 * i + 2, 0, gsem0)
                gath_wait(1, gsem1)
                scat(p, 2 * i + 1, 1)
                return 0
            lax.fori_loop(0, IBLK // 2 - 1, inner, 0)

            gath(p, IBLK - 1, 1, gsem1)
            gath_wait(0, gsem0)
            scat(p, IBLK - 2, 0)
            if b + 1 < NB:
                pltpu.make_async_copy(src_hbm.at[pl.ds(off, IBLK)],
                                      idx_s.at[pn], isem_s).wait()
                pltpu.make_async_copy(dst_hbm.at[pl.ds(off, IBLK)],
                                      idx_d.at[pn], isem_d).wait()
                gath(pn, 0, 0, gsem0)
            gath_wait(1, gsem1)
            scat(p, IBLK - 1, 1)

            if with_counts:
                for _k in range(IBLK):
                    pltpu.make_async_copy(ones_v, cnt_acc.at[idx_d.at[p, 0]],
                                          osem).wait()

        plsc.subcore_barrier()

        @pl.when(s < 15)
        def _():
            pltpu.sync_copy(acc.at[pl.ds(s * CNT_CHUNK, CNT_CHUNK)],
                            out_hbm.at[pl.ds(s * CNT_CHUNK, CNT_CHUNK)])
            if with_counts:
                pltpu.sync_copy(cnt_acc.at[pl.ds(s * CNT_CHUNK, CNT_CHUNK)],
                                zbuf.at[pl.ds(0, CNT_CHUNK)])
                pltpu.sync_copy(zbuf.at[pl.ds(0, CNT_CHUNK)],
                                cnt_out_hbm.at[pl.ds(s * CNT_CHUNK, CNT_CHUNK)])

        @pl.when(s == 15)
        def _():
            pltpu.sync_copy(acc.at[pl.ds(15 * CNT_CHUNK, CNT_LAST)],
                            out_hbm.at[pl.ds(15 * CNT_CHUNK, CNT_LAST)])
            if with_counts:
                pltpu.sync_copy(cnt_acc.at[pl.ds(15 * CNT_CHUNK, CNT_LAST)],
                                zbuf.at[pl.ds(0, CNT_LAST)])
                pltpu.sync_copy(zbuf.at[pl.ds(0, CNT_LAST)],
                                cnt_out_hbm.at[pl.ds(15 * CNT_CHUNK, CNT_LAST)])

    @pl.when(c == 0)
    def _():
        process(src_ui, dst_ui, tab_ui, aggr_i, cnt_i)

    @pl.when(c == 1)
    def _():
        process(src_iu, dst_iu, tab_iu, aggr_u, cnt_u)


def _make_agg(with_counts):
    outs = [jax.ShapeDtypeStruct((N, D), jnp.float32),
            jax.ShapeDtypeStruct((N, D), jnp.float32)]
    if with_counts:
        outs += [jax.ShapeDtypeStruct((N,), jnp.float32),
                 jax.ShapeDtypeStruct((N,), jnp.float32)]
    return pl.kernel(
        functools.partial(_agg_body, with_counts),
        out_type=tuple(outs),
        mesh=_MESH,
        scratch_types=[
            pltpu.VMEM((2, IBLK, CH), jnp.int32),
            pltpu.VMEM((2, IBLK, CH), jnp.int32),
            pltpu.VMEM((2, CH, D), jnp.float32),
            pltpu.VMEM((CH,), jnp.float32),
            pltpu.VMEM((ZERO_PER_TILE,), jnp.float32),
            pltpu.SemaphoreType.DMA,
            pltpu.SemaphoreType.DMA,
            pltpu.SemaphoreType.DMA,
            pltpu.SemaphoreType.DMA,
            pltpu.SemaphoreType.DMA,
            pltpu.VMEM_SHARED((NPAD, D), jnp.float32),
            pltpu.VMEM_SHARED((NPAD,), jnp.float32),
        ],
    )


_agg_counts = _make_agg(True)
_agg_plain = _make_agg(False)


def _dot_t(a, w):
    return lax.dot_general(a, w, (((1,), (1,)), ((), ())),
                           preferred_element_type=jnp.float32)


_BLK = 2000
_GRID = N // _BLK


def _row_spec():
    return pl.BlockSpec((_BLK, D), lambda i: (i, 0))


def _w_spec():
    return pl.BlockSpec((D, D), lambda i: (0, 0))


def _b_spec():
    return pl.BlockSpec((1, D), lambda i: (0, 0))


def _cnt_spec():
    return pl.BlockSpec((_BLK, 1), lambda i: (i, 0))


_ECHUNK = NCHUNK // _GRID
_EROWS = E // CH


def _proj_body(xu, xi, pu, pi, eui, eiu, hu, hi, sui, dui, siu, diu):
    hu[...] = _dot_t(xu[...], pu[...])
    hi[...] = _dot_t(xi[...], pi[...])
    i = pl.program_id(0)
    row = i * _ECHUNK + jax.lax.broadcasted_iota(jnp.int32, (_ECHUNK, CH), 0)
    lane = jax.lax.broadcasted_iota(jnp.int32, (_ECHUNK, CH), 1)
    e = row * CH + lane
    valid = row < _EROWS
    pad_src = (e - E) % N
    pad_dst = N + (e - E) % PAD_DST
    sui[...] = jnp.where(valid, eui[0], pad_src)
    dui[...] = jnp.where(valid, eui[1], pad_dst)
    siu[...] = jnp.where(valid, eiu[0], pad_src)
    diu[...] = jnp.where(valid, eiu[1], pad_dst)


def _e_spec():
    return pl.BlockSpec((2, _ECHUNK, CH), lambda i: (0, i, 0))


def _eo_spec():
    return pl.BlockSpec((_ECHUNK, CH), lambda i: (i, 0))


_proj = pl.pallas_call(
    _proj_body,
    grid=(_GRID,),
    in_specs=[_row_spec(), _row_spec(), _w_spec(), _w_spec(),
              _e_spec(), _e_spec()],
    out_specs=[_row_spec(), _row_spec(),
               _eo_spec(), _eo_spec(), _eo_spec(), _eo_spec()],
    out_shape=[jax.ShapeDtypeStruct((N, D), jnp.float32)] * 2
    + [jax.ShapeDtypeStruct((NCHUNK, CH), jnp.int32)] * 4,
)


def _conv_mid_body(ai, ci, hi, wl_ui, bl_ui, wr_ui,
                   au, cu, hu, wl_iu, bl_iu, wr_iu, oi, ou):
    mean_i = ai[...] * (1.0 / jnp.maximum(ci[...], 1.0))
    oi[...] = jax.nn.relu(_dot_t(mean_i, wl_ui[...]) + bl_ui[...]
                          + _dot_t(hi[...], wr_ui[...])) + hi[...]
    mean_u = au[...] * (1.0 / jnp.maximum(cu[...], 1.0))
    ou[...] = jax.nn.relu(_dot_t(mean_u, wl_iu[...]) + bl_iu[...]
                          + _dot_t(hu[...], wr_iu[...])) + hu[...]


_conv_mid = pl.pallas_call(
    _conv_mid_body,
    grid=(_GRID,),
    in_specs=[_row_spec(), _cnt_spec(), _row_spec(), _w_spec(), _b_spec(),
              _w_spec(),
              _row_spec(), _cnt_spec(), _row_spec(), _w_spec(), _b_spec(),
              _w_spec()],
    out_specs=[_row_spec(), _row_spec()],
    out_shape=[jax.ShapeDtypeStruct((N, D), jnp.float32)] * 2,
)


def _conv_out_body(ai, ci, oi, wl_ui, bl_ui, wr_ui,
                   au, cu, ou, wl_iu, bl_iu, wr_iu, zu, zi):
    mean_i = ai[...] * (1.0 / jnp.maximum(ci[...], 1.0))
    p_i = _dot_t(mean_i, wl_ui[...]) + bl_ui[...] + _dot_t(oi[...], wr_ui[...])
    nrm_i = jnp.sqrt(jnp.sum(p_i * p_i, axis=1, keepdims=True))
    zi[...] = p_i / jnp.maximum(nrm_i, 1e-12)
    mean_u = au[...] * (1.0 / jnp.maximum(cu[...], 1.0))
    p_u = _dot_t(mean_u, wl_iu[...]) + bl_iu[...] + _dot_t(ou[...], wr_iu[...])
    nrm_u = jnp.sqrt(jnp.sum(p_u * p_u, axis=1, keepdims=True))
    zu[...] = p_u / jnp.maximum(nrm_u, 1e-12)


_conv_out = pl.pallas_call(
    _conv_out_body,
    grid=(_GRID,),
    in_specs=[_row_spec(), _cnt_spec(), _row_spec(), _w_spec(), _b_spec(),
              _w_spec(),
              _row_spec(), _cnt_spec(), _row_spec(), _w_spec(), _b_spec(),
              _w_spec()],
    out_specs=[_row_spec(), _row_spec()],
    out_shape=[jax.ShapeDtypeStruct((N, D), jnp.float32)] * 2,
)


def kernel(x_user, x_item, edge_index_ui, edge_index_iu, P_user, P_item,
           c1_ui_Wl, c1_ui_bl, c1_ui_Wr, c1_iu_Wl, c1_iu_bl, c1_iu_Wr,
           c2_ui_Wl, c2_ui_bl, c2_ui_Wr, c2_iu_Wl, c2_iu_bl, c2_iu_Wr):
    h_u, h_i, src_ui, dst_ui, src_iu, dst_iu = _proj(
        x_user, x_item, P_user, P_item,
        edge_index_ui.reshape(2, _EROWS, CH),
        edge_index_iu.reshape(2, _EROWS, CH))

    aggr_i1, aggr_u1, cnt_i, cnt_u = _agg_counts(
        src_ui, dst_ui, src_iu, dst_iu, h_u, h_i)
    cnt_i2 = cnt_i.reshape(N, 1)
    cnt_u2 = cnt_u.reshape(N, 1)

    o_i, o_u = _conv_mid(
        aggr_i1, cnt_i2, h_i, c1_ui_Wl, c1_ui_bl.reshape(1, D), c1_ui_Wr,
        aggr_u1, cnt_u2, h_u, c1_iu_Wl, c1_iu_bl.reshape(1, D), c1_iu_Wr)

    aggr_i2, aggr_u2 = _agg_plain(
        src_ui, dst_ui, src_iu, dst_iu, o_u, o_i)

    out_u, out_i = _conv_out(
        aggr_i2, cnt_i2, o_i, c2_ui_Wl, c2_ui_bl.reshape(1, D), c2_ui_Wr,
        aggr_u2, cnt_u2, o_u, c2_iu_Wl, c2_iu_bl.reshape(1, D), c2_iu_Wr)

    return (out_u, out_i)


# speedup vs baseline: 1.3384x; 1.0036x over previous
import functools

import jax
import jax.numpy as jnp
from jax import lax
from jax.experimental import pallas as pl
from jax.experimental.pallas import tpu as pltpu
from jax.experimental.pallas import tpu_sc as plsc

N = 10000
E = 320000
D = 128

CH = 128
NS = 16
NCHUNK = 2560
PER_TILE = NCHUNK // NS
NPAD = 10240
PAD_DST = NPAD - N

ZERO_PER_TILE = NPAD // NS
CNT_CHUNK = 632
CNT_LAST = N - 15 * CNT_CHUNK

IBLK = 16

_MESH = plsc.VectorSubcoreMesh(core_axis_name="c", subcore_axis_name="s")


def _agg_body(with_counts, *refs):
    if with_counts:
        (src_ui, dst_ui, src_iu, dst_iu, tab_ui, tab_iu,
         aggr_i, aggr_u, cnt_i, cnt_u,
         idx_s, idx_d, rows, ones_v, zbuf,
         gsem0, gsem1, isem_s, isem_d, osem, acc, cnt_acc) = refs
    else:
        (src_ui, dst_ui, src_iu, dst_iu, tab_ui, tab_iu,
         aggr_i, aggr_u,
         idx_s, idx_d, rows, ones_v, zbuf,
         gsem0, gsem1, isem_s, isem_d, osem, acc, cnt_acc) = refs
        cnt_i = cnt_u = None

    c = lax.axis_index("c")
    s = lax.axis_index("s")

    def process(src_hbm, dst_hbm, tab_hbm, out_hbm, cnt_out_hbm):
        zv = jnp.zeros((16,), jnp.float32)
        start = s * PER_TILE

        pltpu.async_copy(src_hbm.at[pl.ds(start, IBLK)], idx_s.at[0], isem_s)
        pltpu.async_copy(dst_hbm.at[pl.ds(start, IBLK)], idx_d.at[0], isem_d)

        def zrow(r, _):
            for j in range(8):
                rows[0, r, pl.ds(j * 16, 16)] = zv
            return 0
        lax.fori_loop(0, CH, zrow, 0)
        for k in range(ZERO_PER_TILE // CH):
            pltpu.async_copy(rows.at[0],
                             acc.at[pl.ds(s * ZERO_PER_TILE + k * CH, CH)],
                             gsem0)
        for k in range(ZERO_PER_TILE // CH):
            pltpu.make_async_copy(
                rows.at[0], acc.at[pl.ds(s * ZERO_PER_TILE, CH)],
                gsem0).wait()
        if with_counts:
            def zflat(i, _):
                zbuf[pl.ds(i * 16, 16)] = zv
                return 0
            lax.fori_loop(0, ZERO_PER_TILE // 16, zflat, 0)
            for j in range(8):
                ones_v[pl.ds(j * 16, 16)] = jnp.ones((16,), jnp.float32)
            pltpu.sync_copy(zbuf,
                            cnt_acc.at[pl.ds(s * ZERO_PER_TILE,
                                             ZERO_PER_TILE)])

        NB = PER_TILE // IBLK

        def gath(p, k, buf, sem):
            return pltpu.async_copy(tab_hbm.at[idx_s.at[p, k]],
                                    rows.at[buf], sem)

        def scat(p, k, buf):
            pltpu.sync_copy(rows.at[buf], acc.at[idx_d.at[p, k]], add=True)
            if with_counts:
                pltpu.async_copy(ones_v, cnt_acc.at[idx_d.at[p, k]], osem,
                                 add=True)

        def gath_wait(buf, sem):
            pltpu.make_async_copy(tab_hbm.at[idx_s.at[0, 0]],
                                  rows.at[buf], sem).wait()

        pltpu.make_async_copy(src_hbm.at[pl.ds(start, IBLK)], idx_s.at[0],
                              isem_s).wait()
        pltpu.make_async_copy(dst_hbm.at[pl.ds(start, IBLK)], idx_d.at[0],
                              isem_d).wait()
        gath(0, 0, 0, gsem0)

        plsc.subcore_barrier()

        for b in range(NB):
            p = b % 2
            pn = (b + 1) % 2
            off = start + (b + 1) * IBLK
            if b + 1 < NB:
                pltpu.async_copy(src_hbm.at[pl.ds(off, IBLK)],
                                 idx_s.at[pn], isem_s)
                pltpu.async_copy(dst_hbm.at[pl.ds(off, IBLK)],
                                 idx_d.at[pn], isem_d)

            def inner(i, _):
                gath(p, 2 * i + 1, 1, gsem1)
                gath_wait(0, gsem0)
                scat(p, 2 * i, 0)
                gath(p, 2 * i + 2, 0, gsem0)
                gath_wait(1, gsem1)
                scat(p, 2 * i + 1, 1)
                return 0
            lax.fori_loop(0, IBLK // 2 - 1, inner, 0)

            gath(p, IBLK - 1, 1, gsem1)
            gath_wait(0, gsem0)
            scat(p, IBLK - 2, 0)
            if b + 1 < NB:
                pltpu.make_async_copy(src_hbm.at[pl.ds(off, IBLK)],
                                      idx_s.at[pn], isem_s).wait()
                pltpu.make_async_copy(dst_hbm.at[pl.ds(off, IBLK)],
                                      idx_d.at[pn], isem_d).wait()
                gath(pn, 0, 0, gsem0)
            gath_wait(1, gsem1)
            scat(p, IBLK - 1, 1)

            if with_counts:
                for _k in range(IBLK):
                    pltpu.make_async_copy(ones_v, cnt_acc.at[idx_d.at[p, 0]],
                                          osem).wait()

        plsc.subcore_barrier()

        @pl.when(s < 15)
        def _():
            pltpu.sync_copy(acc.at[pl.ds(s * CNT_CHUNK, CNT_CHUNK)],
                            out_hbm.at[pl.ds(s * CNT_CHUNK, CNT_CHUNK)])
            if with_counts:
                pltpu.sync_copy(cnt_acc.at[pl.ds(s * CNT_CHUNK, CNT_CHUNK)],
                                zbuf.at[pl.ds(0, CNT_CHUNK)])
                pltpu.sync_copy(zbuf.at[pl.ds(0, CNT_CHUNK)],
                                cnt_out_hbm.at[pl.ds(s * CNT_CHUNK, CNT_CHUNK)])

        @pl.when(s == 15)
        def _():
            pltpu.sync_copy(acc.at[pl.ds(15 * CNT_CHUNK, CNT_LAST)],
                            out_hbm.at[pl.ds(15 * CNT_CHUNK, CNT_LAST)])
            if with_counts:
                pltpu.sync_copy(cnt_acc.at[pl.ds(15 * CNT_CHUNK, CNT_LAST)],
                                zbuf.at[pl.ds(0, CNT_LAST)])
                pltpu.sync_copy(zbuf.at[pl.ds(0, CNT_LAST)],
                                cnt_out_hbm.at[pl.ds(15 * CNT_CHUNK, CNT_LAST)])

    @pl.when(c == 0)
    def _():
        process(src_ui, dst_ui, tab_ui, aggr_i, cnt_i)

    @pl.when(c == 1)
    def _():
        process(src_iu, dst_iu, tab_iu, aggr_u, cnt_u)


def _make_agg(with_counts):
    outs = [jax.ShapeDtypeStruct((N, D), jnp.float32),
            jax.ShapeDtypeStruct((N, D), jnp.float32)]
    if with_counts:
        outs += [jax.ShapeDtypeStruct((N,), jnp.float32),
                 jax.ShapeDtypeStruct((N,), jnp.float32)]
    return pl.kernel(
        functools.partial(_agg_body, with_counts),
        out_type=tuple(outs),
        mesh=_MESH,
        scratch_types=[
            pltpu.VMEM((2, IBLK, CH), jnp.int32),
            pltpu.VMEM((2, IBLK, CH), jnp.int32),
            pltpu.VMEM((2, CH, D), jnp.float32),
            pltpu.VMEM((CH,), jnp.float32),
            pltpu.VMEM((ZERO_PER_TILE,), jnp.float32),
            pltpu.SemaphoreType.DMA,
            pltpu.SemaphoreType.DMA,
            pltpu.SemaphoreType.DMA,
            pltpu.SemaphoreType.DMA,
            pltpu.SemaphoreType.DMA,
            pltpu.VMEM_SHARED((NPAD, D), jnp.float32),
            pltpu.VMEM_SHARED((NPAD,), jnp.float32),
        ],
    )


_agg_counts = _make_agg(True)
_agg_plain = _make_agg(False)


def _dot_t(a, w):
    return lax.dot_general(a, w, (((1,), (1,)), ((), ())),
                           preferred_element_type=jnp.float32)


_BLK = 2000
_GRID = N // _BLK


def _row_spec():
    return pl.BlockSpec((_BLK, D), lambda i: (i, 0))


def _w_spec():
    return pl.BlockSpec((D, D), lambda i: (0, 0))


def _b_spec():
    return pl.BlockSpec((1, D), lambda i: (0, 0))


def _cnt_spec():
    return pl.BlockSpec((_BLK, 1), lambda i: (i, 0))


_ECHUNK = NCHUNK // _GRID
_EROWS = E // CH


def _proj_body(xu, xi, pu, pi, eui, eiu, hu, hi, sui, dui, siu, diu):
    hu[...] = _dot_t(xu[...], pu[...])
    hi[...] = _dot_t(xi[...], pi[...])
    i = pl.program_id(0)
    row = i * _ECHUNK + jax.lax.broadcasted_iota(jnp.int32, (_ECHUNK, CH), 0)
    lane = jax.lax.broadcasted_iota(jnp.int32, (_ECHUNK, CH), 1)
    e = row * CH + lane
    valid = row < _EROWS
    pad_src = (e - E) % N
    pad_dst = N + (e - E) % PAD_DST
    sui[...] = jnp.where(valid, eui[0], pad_src)
    dui[...] = jnp.where(valid, eui[1], pad_dst)
    siu[...] = jnp.where(valid, eiu[0], pad_src)
    diu[...] = jnp.where(valid, eiu[1], pad_dst)


def _e_spec():
    return pl.BlockSpec((2, _ECHUNK, CH), lambda i: (0, i, 0))


def _eo_spec():
    return pl.BlockSpec((_ECHUNK, CH), lambda i: (i, 0))


_proj = pl.pallas_call(
    _proj_body,
    grid=(_GRID,),
    in_specs=[_row_spec(), _row_spec(), _w_spec(), _w_spec(),
              _e_spec(), _e_spec()],
    out_specs=[_row_spec(), _row_spec(),
               _eo_spec(), _eo_spec(), _eo_spec(), _eo_spec()],
    out_shape=[jax.ShapeDtypeStruct((N, D), jnp.float32)] * 2
    + [jax.ShapeDtypeStruct((NCHUNK, CH), jnp.int32)] * 4,
)


def _conv_mid_body(ai, ci, hi, wl_ui, bl_ui, wr_ui,
                   au, cu, hu, wl_iu, bl_iu, wr_iu, oi, ou):
    mean_i = ai[...] * (1.0 / jnp.maximum(ci[...], 1.0))
    oi[...] = jax.nn.relu(_dot_t(mean_i, wl_ui[...]) + bl_ui[...]
                          + _dot_t(hi[...], wr_ui[...])) + hi[...]
    mean_u = au[...] * (1.0 / jnp.maximum(cu[...], 1.0))
    ou[...] = jax.nn.relu(_dot_t(mean_u, wl_iu[...]) + bl_iu[...]
                          + _dot_t(hu[...], wr_iu[...])) + hu[...]


_conv_mid = pl.pallas_call(
    _conv_mid_body,
    grid=(_GRID,),
    in_specs=[_row_spec(), _cnt_spec(), _row_spec(), _w_spec(), _b_spec(),
              _w_spec(),
              _row_spec(), _cnt_spec(), _row_spec(), _w_spec(), _b_spec(),
              _w_spec()],
    out_specs=[_row_spec(), _row_spec()],
    out_shape=[jax.ShapeDtypeStruct((N, D), jnp.float32)] * 2,
)


def _conv_out_body(ai, ci, oi, wl_ui, bl_ui, wr_ui,
                   au, cu, ou, wl_iu, bl_iu, wr_iu, zu, zi):
    mean_i = ai[...] * (1.0 / jnp.maximum(ci[...], 1.0))
    p_i = _dot_t(mean_i, wl_ui[...]) + bl_ui[...] + _dot_t(oi[...], wr_ui[...])
    nrm_i = jnp.sqrt(jnp.sum(p_i * p_i, axis=1, keepdims=True))
    zi[...] = p_i / jnp.maximum(nrm_i, 1e-12)
    mean_u = au[...] * (1.0 / jnp.maximum(cu[...], 1.0))
    p_u = _dot_t(mean_u, wl_iu[...]) + bl_iu[...] + _dot_t(ou[...], wr_iu[...])
    nrm_u = jnp.sqrt(jnp.sum(p_u * p_u, axis=1, keepdims=True))
    zu[...] = p_u / jnp.maximum(nrm_u, 1e-12)


_conv_out = pl.pallas_call(
    _conv_out_body,
    grid=(_GRID,),
    in_specs=[_row_spec(), _cnt_spec(), _row_spec(), _w_spec(), _b_spec(),
              _w_spec(),
              _row_spec(), _cnt_spec(), _row_spec(), _w_spec(), _b_spec(),
              _w_spec()],
    out_specs=[_row_spec(), _row_spec()],
    out_shape=[jax.ShapeDtypeStruct((N, D), jnp.float32)] * 2,
)


def kernel(x_user, x_item, edge_index_ui, edge_index_iu, P_user, P_item,
           c1_ui_Wl, c1_ui_bl, c1_ui_Wr, c1_iu_Wl, c1_iu_bl, c1_iu_Wr,
           c2_ui_Wl, c2_ui_bl, c2_ui_Wr, c2_iu_Wl, c2_iu_bl, c2_iu_Wr):
    h_u, h_i, src_ui, dst_ui, src_iu, dst_iu = _proj(
        x_user, x_item, P_user, P_item,
        edge_index_ui.reshape(2, _EROWS, CH),
        edge_index_iu.reshape(2, _EROWS, CH))

    aggr_i1, aggr_u1, cnt_i, cnt_u = _agg_counts(
        src_ui, dst_ui, src_iu, dst_iu, h_u, h_i)
    cnt_i2 = cnt_i.reshape(N, 1)
    cnt_u2 = cnt_u.reshape(N, 1)

    o_i, o_u = _conv_mid(
        aggr_i1, cnt_i2, h_i, c1_ui_Wl, c1_ui_bl.reshape(1, D), c1_ui_Wr,
        aggr_u1, cnt_u2, h_u, c1_iu_Wl, c1_iu_bl.reshape(1, D), c1_iu_Wr)

    aggr_i2, aggr_u2 = _agg_plain(
        src_ui, dst_ui, src_iu, dst_iu, o_u, o_i)

    out_u, out_i = _conv_out(
        aggr_i2, cnt_i2, o_i, c2_ui_Wl, c2_ui_bl.reshape(1, D), c2_ui_Wr,
        aggr_u2, cnt_u2, o_u, c2_iu_Wl, c2_iu_bl.reshape(1, D), c2_iu_Wr)

    return (out_u, out_i)
